# baseline (speedup 1.0000x reference)
import functools

import jax
import jax.numpy as jnp
from jax import lax
from jax.experimental import pallas as pl
from jax.experimental.pallas import tpu as pltpu, tpu_sc as plsc

_CHUNK = 128


def kernel(node_ids_user, node_ids_item, table_user, table_item):
    B = node_ids_user.shape[0]
    D = table_user.shape[1]
    info = plsc.get_sparse_core_info()
    NW = info.num_cores * info.num_subcores
    b_per_w = B // NW
    nchunk = b_per_w // _CHUNK

    idx_u = jnp.reshape(node_ids_user, (NW, nchunk, _CHUNK))
    idx_i = jnp.reshape(node_ids_item, (NW, nchunk, _CHUNK))

    mesh = plsc.VectorSubcoreMesh(core_axis_name="c", subcore_axis_name="s")

    @functools.partial(
        pl.kernel,
        mesh=mesh,
        out_type=(
            jax.ShapeDtypeStruct((NW, nchunk, _CHUNK, D), jnp.float32),
            jax.ShapeDtypeStruct((NW, nchunk, _CHUNK, D), jnp.float32),
        ),
        scratch_types=[
            pltpu.VMEM((nchunk, _CHUNK), jnp.int32),
            pltpu.VMEM((nchunk, _CHUNK), jnp.int32),
            pltpu.VMEM((nchunk, _CHUNK, D), jnp.float32),
            pltpu.VMEM((nchunk, _CHUNK, D), jnp.float32),
            pltpu.SemaphoreType.DMA,
            pltpu.SemaphoreType.DMA,
        ],
        compiler_params=pltpu.CompilerParams(use_tc_tiling_on_sc=False),
    )
    def k(idx_u_hbm, idx_i_hbm, tab_u_hbm, tab_i_hbm, out_u_hbm, out_i_hbm,
          idx_u_v, idx_i_v, rows_u_v, rows_i_v, sem_u, sem_i):
        wid = lax.axis_index("s") * info.num_cores + lax.axis_index("c")
        pltpu.sync_copy(idx_u_hbm.at[wid], idx_u_v)
        pltpu.sync_copy(idx_i_hbm.at[wid], idx_i_v)
        copies = []
        for c in range(nchunk):
            copies.append(
                pltpu.async_copy(tab_u_hbm.at[idx_u_v.at[c]], rows_u_v.at[c], sem_u))
            copies.append(
                pltpu.async_copy(tab_i_hbm.at[idx_i_v.at[c]], rows_i_v.at[c], sem_i))
        for cp in copies:
            cp.wait()
        pltpu.sync_copy(rows_u_v, out_u_hbm.at[wid])
        pltpu.sync_copy(rows_i_v, out_i_hbm.at[wid])

    out_u, out_i = k(idx_u, idx_i, table_user, table_item)
    return (jnp.reshape(out_u, (B, D)), jnp.reshape(out_i, (B, D)))


# speedup vs baseline: 2.3965x; 2.3965x over previous
import functools

import jax
import jax.numpy as jnp
from jax import lax
from jax.experimental import pallas as pl
from jax.experimental.pallas import tpu as pltpu, tpu_sc as plsc

_RING = 8
_ICHUNK = 128


def _user_lookup(node_ids_user, tab_t, B, D, NW, num_cores):
    b_per_w = B // NW
    nvec = b_per_w // 16
    idx3 = jnp.reshape(node_ids_user, (NW, nvec, 16))
    mesh = plsc.VectorSubcoreMesh(core_axis_name="c", subcore_axis_name="s")

    @functools.partial(
        pl.kernel,
        mesh=mesh,
        out_type=jax.ShapeDtypeStruct((D, B), jnp.float32),
        scratch_types=[
            pltpu.VMEM((nvec, 16), jnp.int32),
            pltpu.VMEM((_RING, D, 128), jnp.float32),
            pltpu.VMEM((D, b_per_w), jnp.float32),
        ] + [pltpu.SemaphoreType.DMA] * _RING,
        compiler_params=pltpu.CompilerParams(use_tc_tiling_on_sc=True,
                                             needs_layout_passes=False),
    )
    def k(idx_hbm, tab_hbm, out_hbm, idx_v, ring_v, out_v, *sems):
        wid = lax.axis_index("s") * num_cores + lax.axis_index("c")
        pltpu.sync_copy(idx_hbm.at[wid], idx_v)
        lanes = lax.iota(jnp.int32, 16)

        def lane_of(vec, lane):
            return jnp.max(jnp.where(lanes == lane, vec, 0))

        def issue(b, vec, lane):
            jv = lane_of(vec, lane)
            off = pl.multiple_of((jv >> 7) * 128, 128)
            pltpu.async_copy(tab_hbm.at[:, pl.ds(off, 128)], ring_v.at[b],
                             sems[b])

        vec0 = idx_v[0, pl.ds(0, 16)]
        for b in range(_RING):
            issue(b, vec0, b)

        def body(c, carry):
            vec_cur = idx_v[c >> 1, pl.ds(0, 16)]
            cn = jnp.minimum(c + 1, 2 * nvec - 1)
            vec_nxt = idx_v[cn >> 1, pl.ds(0, 16)]
            lane_cur = (c & 1) * 8
            lane_nxt = (cn & 1) * 8
            for b in range(_RING):
                jv = lane_of(vec_cur, lane_cur + b)
                col = jv & 127
                pltpu.make_async_copy(tab_hbm.at[:, pl.ds(0, 128)],
                                      ring_v.at[b], sems[b]).wait()
                outpos = c * 8 + b
                for kq in range(D // 16):
                    dvec = lanes + kq * 16
                    vals = plsc.load_gather(
                        ring_v.at[b], [dvec, jnp.full((16,), 1, jnp.int32) * col])
                    plsc.store_scatter(
                        out_v, [dvec, jnp.full((16,), 1, jnp.int32) * outpos],
                        vals)
                issue(b, vec_nxt, lane_nxt + b)
            return carry

        lax.fori_loop(0, 2 * nvec, body, 0)
        for b in range(_RING):
            pltpu.make_async_copy(tab_hbm.at[:, pl.ds(0, 128)], ring_v.at[b],
                                  sems[b]).wait()
        out_off = pl.multiple_of(wid * b_per_w, 128)
        pltpu.sync_copy(out_v, out_hbm.at[:, pl.ds(out_off, b_per_w)])

    return k(idx3, tab_t)


def _item_lookup(node_ids_item, table_item, B, D, NW, num_cores):
    b_per_w = B // NW
    nchunk = b_per_w // _ICHUNK
    idx3 = jnp.reshape(node_ids_item, (NW, nchunk, _ICHUNK))
    mesh = plsc.VectorSubcoreMesh(core_axis_name="c", subcore_axis_name="s")

    @functools.partial(
        pl.kernel,
        mesh=mesh,
        out_type=jax.ShapeDtypeStruct((NW, nchunk, _ICHUNK, D), jnp.float32),
        scratch_types=[
            pltpu.VMEM((nchunk, _ICHUNK), jnp.int32),
            pltpu.VMEM((nchunk, _ICHUNK, D), jnp.float32),
            pltpu.SemaphoreType.DMA,
        ],
        compiler_params=pltpu.CompilerParams(use_tc_tiling_on_sc=False),
    )
    def k(idx_hbm, tab_hbm, out_hbm, idx_v, rows_v, sem):
        wid = lax.axis_index("s") * num_cores + lax.axis_index("c")
        pltpu.sync_copy(idx_hbm.at[wid], idx_v)
        copies = [
            pltpu.async_copy(tab_hbm.at[idx_v.at[c]], rows_v.at[c], sem)
            for c in range(nchunk)
        ]
        for cp in copies:
            cp.wait()
        pltpu.sync_copy(rows_v, out_hbm.at[wid])

    out = k(idx3, table_item)
    return jnp.reshape(out, (B, D))


def kernel(node_ids_user, node_ids_item, table_user, table_item):
    B = node_ids_user.shape[0]
    D = table_user.shape[1]
    info = plsc.get_sparse_core_info()
    NW = info.num_cores * info.num_subcores
    tab_t = table_user.T
    emb_u_t = _user_lookup(node_ids_user, tab_t, B, D, NW, info.num_cores)
    emb_i = _item_lookup(node_ids_item, table_item, B, D, NW, info.num_cores)
    return (emb_u_t.T, emb_i)


# speedup vs baseline: 2.4704x; 1.0308x over previous
import functools

import jax
import jax.numpy as jnp
from jax import lax
from jax.experimental import pallas as pl
from jax.experimental.pallas import tpu as pltpu, tpu_sc as plsc

_RING = 8
_ICHUNK = 128


def _user_lookup(node_ids_user, tab_t, B, D, NW, num_cores):
    b_per_w = B // NW
    nvec = b_per_w // 16
    idx3 = jnp.reshape(node_ids_user, (NW, nvec, 16))
    mesh = plsc.VectorSubcoreMesh(core_axis_name="c", subcore_axis_name="s")

    @functools.partial(
        pl.kernel,
        mesh=mesh,
        out_type=jax.ShapeDtypeStruct((D, B), jnp.float32),
        scratch_types=[
            pltpu.VMEM((nvec, 16), jnp.int32),
            pltpu.VMEM((_RING, D, 128), jnp.float32),
            pltpu.VMEM((D, b_per_w), jnp.float32),
        ] + [pltpu.SemaphoreType.DMA] * _RING,
        compiler_params=pltpu.CompilerParams(use_tc_tiling_on_sc=True,
                                             needs_layout_passes=False),
    )
    def k(idx_hbm, tab_hbm, out_hbm, idx_v, ring_v, out_v, *sems):
        wid = lax.axis_index("s") * num_cores + lax.axis_index("c")
        pltpu.sync_copy(idx_hbm.at[wid], idx_v)
        lanes = lax.iota(jnp.int32, 16)

        def lane_of(vec, lane):
            return jnp.max(jnp.where(lanes == lane, vec, 0))

        def issue(b, vec, lane):
            jv = lane_of(vec, lane)
            off = pl.multiple_of((jv >> 7) * 128, 128)
            pltpu.async_copy(tab_hbm.at[:, pl.ds(off, 128)], ring_v.at[b],
                             sems[b])

        vec0 = idx_v[0, pl.ds(0, 16)]
        for b in range(_RING):
            issue(b, vec0, b)

        def body(c, carry):
            vec_cur = idx_v[c >> 1, pl.ds(0, 16)]
            cn = jnp.minimum(c + 1, 2 * nvec - 1)
            vec_nxt = idx_v[cn >> 1, pl.ds(0, 16)]
            lane_cur = (c & 1) * 8
            lane_nxt = (cn & 1) * 8
            for b in range(_RING):
                jv = lane_of(vec_cur, lane_cur + b)
                col = jv & 127
                pltpu.make_async_copy(tab_hbm.at[:, pl.ds(0, 128)],
                                      ring_v.at[b], sems[b]).wait()
                outpos = c * 8 + b
                for kq in range(D // 16):
                    dvec = lanes + kq * 16
                    vals = plsc.load_gather(
                        ring_v.at[b], [dvec, jnp.full((16,), 1, jnp.int32) * col])
                    plsc.store_scatter(
                        out_v, [dvec, jnp.full((16,), 1, jnp.int32) * outpos],
                        vals)
                issue(b, vec_nxt, lane_nxt + b)
            return carry

        lax.fori_loop(0, 2 * nvec, body, 0)
        for b in range(_RING):
            pltpu.make_async_copy(tab_hbm.at[:, pl.ds(0, 128)], ring_v.at[b],
                                  sems[b]).wait()
        out_off = pl.multiple_of(wid * b_per_w, 128)
        pltpu.sync_copy(out_v, out_hbm.at[:, pl.ds(out_off, b_per_w)])

    return k(idx3, tab_t)


def _item_lookup(node_ids_item, table_item, B, D, NW, num_cores):
    b_per_w = B // NW
    nchunk = b_per_w // _ICHUNK
    idx3 = jnp.reshape(node_ids_item, (NW, nchunk, _ICHUNK))
    mesh = plsc.VectorSubcoreMesh(core_axis_name="c", subcore_axis_name="s")

    @functools.partial(
        pl.kernel,
        mesh=mesh,
        out_type=jax.ShapeDtypeStruct((NW, nchunk, _ICHUNK, D), jnp.float32),
        scratch_types=[
            pltpu.VMEM((nchunk, _ICHUNK), jnp.int32),
            pltpu.VMEM((nchunk, _ICHUNK, D), jnp.float32),
            pltpu.SemaphoreType.DMA,
        ],
        compiler_params=pltpu.CompilerParams(use_tc_tiling_on_sc=False),
    )
    def k(idx_hbm, tab_hbm, out_hbm, idx_v, rows_v, sem):
        wid = lax.axis_index("s") * num_cores + lax.axis_index("c")
        pltpu.sync_copy(idx_hbm.at[wid], idx_v)
        copies = [
            pltpu.async_copy(tab_hbm.at[idx_v.at[c]], rows_v.at[c], sem)
            for c in range(nchunk)
        ]
        for cp in copies:
            cp.wait()
        pltpu.sync_copy(rows_v, out_hbm.at[wid])

    out = k(idx3, table_item)
    return jnp.reshape(out, (B, D))


def kernel(node_ids_user, node_ids_item, table_user, table_item):
    B = node_ids_user.shape[0]
    D = table_user.shape[1]
    info = plsc.get_sparse_core_info()
    NW = info.num_cores * info.num_subcores
    tab_t = table_user.T
    emb_u_t = _user_lookup(node_ids_user, tab_t, B, D, NW, info.num_cores)
    node_ids_item, emb_u_t = lax.optimization_barrier((node_ids_item, emb_u_t))
    emb_i = _item_lookup(node_ids_item, table_item, B, D, NW, info.num_cores)
    return (emb_u_t.T, emb_i)
